# baseline (speedup 1.0000x reference)
import functools

import jax
import jax.numpy as jnp
from jax import lax
from jax.experimental import pallas as pl
from jax.experimental.pallas import tpu as pltpu
from jax.experimental.pallas import tpu_sc as plsc

D = 768
HALF = D // 2
DFF = 2048
N_TOK = 1024 * 77
NUM_WORKERS = 32
PER_W = N_TOK // NUM_WORKERS
CHUNK = 56
NCHUNK = PER_W // CHUNK
LANES = 16
D_SLICES = D // LANES


def _mlp_kernel(t_ref, w1_ref, b1_ref, w2_ref, b2_ref, e_ref):
    t = t_ref[0].astype(jnp.float32)
    col = lax.broadcasted_iota(jnp.int32, (1, D), 1).astype(jnp.float32)
    idx = jnp.where(col < HALF, col, col - HALF)
    freqs = jnp.exp(idx * (-jnp.log(jnp.float32(10000.0)) / HALF))
    args = t * freqs
    temb = jnp.where(col < HALF, jnp.cos(args), jnp.sin(args))
    h = lax.dot_general(temb, w1_ref[...], (((1,), (1,)), ((), ())),
                        preferred_element_type=jnp.float32)
    h = h + b1_ref[...][None, :]
    h = h * (1.0 / (1.0 + jnp.exp(-h)))
    e = lax.dot_general(h, w2_ref[...], (((1,), (1,)), ((), ())),
                        preferred_element_type=jnp.float32)
    e_ref[...] = e + b2_ref[...][None, :]


def _compute_e(timesteps, W1, b1, W2, b2):
    return pl.pallas_call(
        _mlp_kernel,
        out_shape=jax.ShapeDtypeStruct((1, D), jnp.float32),
        in_specs=[
            pl.BlockSpec(memory_space=pltpu.SMEM),
            pl.BlockSpec(memory_space=pltpu.VMEM),
            pl.BlockSpec(memory_space=pltpu.VMEM),
            pl.BlockSpec(memory_space=pltpu.VMEM),
            pl.BlockSpec(memory_space=pltpu.VMEM),
        ],
        out_specs=pl.BlockSpec(memory_space=pltpu.VMEM),
    )(timesteps, W1, b1, W2, b2)


def _gather_scale(table_hbm, idx_hbm, e_hbm, out_hbm, idx_v, rows_v, e_v, sem):
    wid = lax.axis_index("s") * 2 + lax.axis_index("c")
    base = wid * PER_W
    pltpu.sync_copy(e_hbm, e_v)

    @pl.loop(0, NCHUNK)
    def _chunk(ch):
        start = base + ch * CHUNK
        pltpu.sync_copy(idx_hbm.at[pl.ds(start, CHUNK)], idx_v)
        pltpu.async_copy(table_hbm.at[idx_v], rows_v, sem).wait()

        @pl.loop(0, CHUNK)
        def _row(r):
            for d in range(D_SLICES):
                sl = pl.ds(d * LANES, LANES)
                rows_v[r, sl] = rows_v[r, sl] * e_v[sl]

        pltpu.sync_copy(rows_v, out_hbm.at[pl.ds(start, CHUNK)])


def kernel(input_ids, timesteps, W1, b1, W2, b2, table):
    B, L = input_ids.shape
    e = _compute_e(timesteps.astype(jnp.int32), W1, b1, W2, b2)
    ids = input_ids.reshape(-1).astype(jnp.int32)

    mesh = plsc.VectorSubcoreMesh(core_axis_name="c", subcore_axis_name="s")
    sc_kernel = functools.partial(
        pl.kernel,
        mesh=mesh,
        out_type=jax.ShapeDtypeStruct((N_TOK, D), jnp.float32),
        scratch_types=[
            pltpu.VMEM((CHUNK,), jnp.int32),
            pltpu.VMEM((CHUNK, D), jnp.float32),
            pltpu.VMEM((D,), jnp.float32),
            pltpu.SemaphoreType.DMA,
        ],
    )(_gather_scale)
    out = sc_kernel(table, ids, e.reshape(D))
    return out.reshape(B, L, D)


# speedup vs baseline: 1.1251x; 1.1251x over previous
import functools

import jax
import jax.numpy as jnp
from jax import lax
from jax.experimental import pallas as pl
from jax.experimental.pallas import tpu as pltpu
from jax.experimental.pallas import tpu_sc as plsc

D = 768
HALF = D // 2
DFF = 2048
N_TOK = 1024 * 77
NUM_WORKERS = 32
PER_W = N_TOK // NUM_WORKERS
CHUNK = 56
NCHUNK = PER_W // CHUNK
LANES = 16
D_SLICES = D // LANES


def _mlp_kernel(t_ref, w1_ref, b1_ref, w2_ref, b2_ref, e_ref):
    t = t_ref[0].astype(jnp.float32)
    col = lax.broadcasted_iota(jnp.int32, (1, D), 1).astype(jnp.float32)
    idx = jnp.where(col < HALF, col, col - HALF)
    freqs = jnp.exp(idx * (-jnp.log(jnp.float32(10000.0)) / HALF))
    args = t * freqs
    temb = jnp.where(col < HALF, jnp.cos(args), jnp.sin(args))
    h = lax.dot_general(temb, w1_ref[...], (((1,), (1,)), ((), ())),
                        preferred_element_type=jnp.float32)
    h = h + b1_ref[...][None, :]
    h = h * (1.0 / (1.0 + jnp.exp(-h)))
    e = lax.dot_general(h, w2_ref[...], (((1,), (1,)), ((), ())),
                        preferred_element_type=jnp.float32)
    e_ref[...] = e + b2_ref[...][None, :]


def _compute_e(timesteps, W1, b1, W2, b2):
    return pl.pallas_call(
        _mlp_kernel,
        out_shape=jax.ShapeDtypeStruct((1, D), jnp.float32),
        in_specs=[
            pl.BlockSpec(memory_space=pltpu.SMEM),
            pl.BlockSpec(memory_space=pltpu.VMEM),
            pl.BlockSpec(memory_space=pltpu.VMEM),
            pl.BlockSpec(memory_space=pltpu.VMEM),
            pl.BlockSpec(memory_space=pltpu.VMEM),
        ],
        out_specs=pl.BlockSpec(memory_space=pltpu.VMEM),
    )(timesteps, W1, b1, W2, b2)


def _gather_scale(table_hbm, idx_hbm, e_hbm, out_hbm,
                  idx_v, rows0, rows1, e_v, g0, g1, s0, s1):
    wid = lax.axis_index("s") * 2 + lax.axis_index("c")
    base = wid * PER_W
    pltpu.sync_copy(e_hbm, e_v)
    pltpu.sync_copy(idx_hbm.at[pl.ds(base, PER_W)], idx_v)

    rows = (rows0, rows1)
    gsem = (g0, g1)
    ssem = (s0, s1)

    def start_gather(ch, b):
        pltpu.async_copy(
            table_hbm.at[idx_v.at[pl.ds(ch * CHUNK, CHUNK)]], rows[b], gsem[b])

    def wait_gather(b):
        pltpu.make_async_copy(
            table_hbm.at[idx_v.at[pl.ds(0, CHUNK)]], rows[b], gsem[b]).wait()

    def start_store(ch, b):
        pltpu.async_copy(
            rows[b], out_hbm.at[pl.ds(base + ch * CHUNK, CHUNK)], ssem[b])

    def wait_store(b):
        pltpu.make_async_copy(
            rows[b], out_hbm.at[pl.ds(base, CHUNK)], ssem[b]).wait()

    def scale(b):
        @pl.loop(0, CHUNK)
        def _row(r):
            for d in range(D_SLICES):
                sl = pl.ds(d * LANES, LANES)
                rows[b][r, sl] = rows[b][r, sl] * e_v[sl]

    start_gather(0, 0)
    start_gather(1, 1)

    @pl.loop(0, NCHUNK - 2, step=2)
    def _pair(p):
        for b in range(2):
            wait_gather(b)
            scale(b)
            start_store(p + b, b)
        for b in range(2):
            wait_store(b)
            start_gather(p + 2 + b, b)

    for b in range(2):
        wait_gather(b)
        scale(b)
        start_store(NCHUNK - 2 + b, b)
    for b in range(2):
        wait_store(b)


def kernel(input_ids, timesteps, W1, b1, W2, b2, table):
    B, L = input_ids.shape
    e = _compute_e(timesteps.astype(jnp.int32), W1, b1, W2, b2)
    ids = input_ids.reshape(-1).astype(jnp.int32)

    mesh = plsc.VectorSubcoreMesh(core_axis_name="c", subcore_axis_name="s")
    sc_kernel = functools.partial(
        pl.kernel,
        mesh=mesh,
        out_type=jax.ShapeDtypeStruct((N_TOK, D), jnp.float32),
        scratch_types=[
            pltpu.VMEM((PER_W,), jnp.int32),
            pltpu.VMEM((CHUNK, D), jnp.float32),
            pltpu.VMEM((CHUNK, D), jnp.float32),
            pltpu.VMEM((D,), jnp.float32),
            pltpu.SemaphoreType.DMA,
            pltpu.SemaphoreType.DMA,
            pltpu.SemaphoreType.DMA,
            pltpu.SemaphoreType.DMA,
        ],
    )(_gather_scale)
    out = sc_kernel(table, ids, e.reshape(D))
    return out.reshape(B, L, D)


# speedup vs baseline: 2.0334x; 1.8072x over previous
import functools

import jax
import jax.numpy as jnp
from jax import lax
from jax.experimental import pallas as pl
from jax.experimental.pallas import tpu as pltpu
from jax.experimental.pallas import tpu_sc as plsc

D = 768
HALF = D // 2
DFF = 2048
N_TOK = 1024 * 77
NUM_WORKERS = 32
PER_W = N_TOK // NUM_WORKERS
CHUNK = 56
NCHUNK = PER_W // CHUNK
LANES = 16
D_SLICES = D // LANES


def _mlp_kernel(t_ref, w1_ref, b1_ref, w2_ref, b2_ref, e_ref):
    t = t_ref[0].astype(jnp.float32)
    col = lax.broadcasted_iota(jnp.int32, (1, D), 1).astype(jnp.float32)
    idx = jnp.where(col < HALF, col, col - HALF)
    freqs = jnp.exp(idx * (-jnp.log(jnp.float32(10000.0)) / HALF))
    args = t * freqs
    temb = jnp.where(col < HALF, jnp.cos(args), jnp.sin(args))
    h = lax.dot_general(temb, w1_ref[...], (((1,), (1,)), ((), ())),
                        preferred_element_type=jnp.float32)
    h = h + b1_ref[...][None, :]
    h = h * (1.0 / (1.0 + jnp.exp(-h)))
    e = lax.dot_general(h, w2_ref[...], (((1,), (1,)), ((), ())),
                        preferred_element_type=jnp.float32)
    e_ref[...] = e + b2_ref[...][None, :]


def _compute_e(timesteps, W1, b1, W2, b2):
    return pl.pallas_call(
        _mlp_kernel,
        out_shape=jax.ShapeDtypeStruct((1, D), jnp.float32),
        in_specs=[
            pl.BlockSpec(memory_space=pltpu.SMEM),
            pl.BlockSpec(memory_space=pltpu.VMEM),
            pl.BlockSpec(memory_space=pltpu.VMEM),
            pl.BlockSpec(memory_space=pltpu.VMEM),
            pl.BlockSpec(memory_space=pltpu.VMEM),
        ],
        out_specs=pl.BlockSpec(memory_space=pltpu.VMEM),
    )(timesteps, W1, b1, W2, b2)


def _gather_scale(table_hbm, idx_hbm, e_hbm, out_hbm,
                  idx_v, rows0, rows1, e_v, g0, g1, s0, s1):
    wid = lax.axis_index("s") * 2 + lax.axis_index("c")
    base = wid * PER_W
    pltpu.sync_copy(e_hbm, e_v)
    pltpu.sync_copy(idx_hbm.at[pl.ds(base, PER_W)], idx_v)

    rows = (rows0, rows1)
    gsem = (g0, g1)
    ssem = (s0, s1)

    def start_gather(ch, b):
        pltpu.async_copy(
            table_hbm.at[idx_v.at[pl.ds(ch * CHUNK, CHUNK)]], rows[b], gsem[b])

    def wait_gather(b):
        pltpu.make_async_copy(
            table_hbm.at[idx_v.at[pl.ds(0, CHUNK)]], rows[b], gsem[b]).wait()

    def start_store(ch, b):
        pltpu.async_copy(
            rows[b], out_hbm.at[pl.ds(base + ch * CHUNK, CHUNK)], ssem[b])

    def wait_store(b):
        pltpu.make_async_copy(
            rows[b], out_hbm.at[pl.ds(base, CHUNK)], ssem[b]).wait()

    def scale(b):
        for band in range(D_SLICES // 8):
            e_regs = [e_v[pl.ds((band * 8 + j) * LANES, LANES)]
                      for j in range(8)]

            @pl.loop(0, CHUNK)
            def _row(r):
                for j in range(8):
                    sl = pl.ds((band * 8 + j) * LANES, LANES)
                    rows[b][r, sl] = rows[b][r, sl] * e_regs[j]

    start_gather(0, 0)
    start_gather(1, 1)

    @pl.loop(0, NCHUNK - 2, step=2)
    def _pair(p):
        for b in range(2):
            wait_gather(b)
            scale(b)
            start_store(p + b, b)
        for b in range(2):
            wait_store(b)
            start_gather(p + 2 + b, b)

    for b in range(2):
        wait_gather(b)
        scale(b)
        start_store(NCHUNK - 2 + b, b)
    for b in range(2):
        wait_store(b)


def kernel(input_ids, timesteps, W1, b1, W2, b2, table):
    B, L = input_ids.shape
    e = _compute_e(timesteps.astype(jnp.int32), W1, b1, W2, b2)
    ids = input_ids.reshape(-1).astype(jnp.int32)

    mesh = plsc.VectorSubcoreMesh(core_axis_name="c", subcore_axis_name="s")
    sc_kernel = functools.partial(
        pl.kernel,
        mesh=mesh,
        out_type=jax.ShapeDtypeStruct((N_TOK, D), jnp.float32),
        scratch_types=[
            pltpu.VMEM((PER_W,), jnp.int32),
            pltpu.VMEM((CHUNK, D), jnp.float32),
            pltpu.VMEM((CHUNK, D), jnp.float32),
            pltpu.VMEM((D,), jnp.float32),
            pltpu.SemaphoreType.DMA,
            pltpu.SemaphoreType.DMA,
            pltpu.SemaphoreType.DMA,
            pltpu.SemaphoreType.DMA,
        ],
    )(_gather_scale)
    out = sc_kernel(table, ids, e.reshape(D))
    return out.reshape(B, L, D)
